# baseline (speedup 1.0000x reference)
import jax
import jax.numpy as jnp
from jax.experimental import pallas as pl
from jax.experimental.pallas import tpu as pltpu

OUT_DIM = 32
IN_DIM = 50000
BATCH = 4096
_TAU = 10.0 * (0.01 / 10.0) ** (1.0 / 10000.0)

KB = 1000
KSTEPS = IN_DIM // KB

CH = 5120
_CHUNKS = [(o, min(CH, IN_DIM - o)) for o in range(0, IN_DIM, CH)]


def _fused_kernel(xt_ref, xt2_ref, a_hbm, g_hbm, out_ref,
                  acc_ref, abuf, gbuf, mt_ref, sema, semg):
    k = pl.program_id(0)

    @pl.when(k == 0)
    def _softmax():
        cpa = pltpu.make_async_copy(a_hbm, abuf, sema)
        cpg = pltpu.make_async_copy(g_hbm, gbuf, semg)
        cpa.start()
        cpg.start()
        cpa.wait()
        cpg.wait()
        s = jnp.zeros((OUT_DIM, 1), jnp.float32)
        for off, w in _CHUNKS:
            e = jnp.exp((abuf[:, off:off + w] + gbuf[:, off:off + w])
                        * (1.0 / _TAU))
            abuf[:, off:off + w] = e
            s = s + jnp.sum(e, axis=1, keepdims=True)
        rs = 1.0 / s
        for off, w in _CHUNKS:
            mt_ref[off:off + w, :] = (
                abuf[:, off:off + w] * rs).astype(jnp.bfloat16).T

    m = mt_ref[pl.ds(k * KB, KB), :].T
    part1 = jnp.dot(m, xt_ref[...].astype(jnp.bfloat16),
                    preferred_element_type=jnp.float32)
    part2 = jnp.dot(m, xt2_ref[...].astype(jnp.bfloat16),
                    preferred_element_type=jnp.float32)

    @pl.when(k == 0)
    def _init():
        acc_ref[:, :BATCH // 2] = part1
        acc_ref[:, BATCH // 2:] = part2

    @pl.when(k != 0)
    def _acc():
        acc_ref[:, :BATCH // 2] += part1
        acc_ref[:, BATCH // 2:] += part2

    @pl.when(k == KSTEPS - 1)
    def _fin():
        out_ref[...] = acc_ref[...]


def kernel(x, alphas, gumbel):
    xt = jnp.transpose(x)

    out_t = pl.pallas_call(
        _fused_kernel,
        grid=(KSTEPS,),
        in_specs=[
            pl.BlockSpec((KB, BATCH // 2), lambda k: (k, 0)),
            pl.BlockSpec((KB, BATCH // 2), lambda k: (k, 1)),
            pl.BlockSpec(memory_space=pltpu.HBM),
            pl.BlockSpec(memory_space=pltpu.HBM),
        ],
        out_specs=pl.BlockSpec((OUT_DIM, BATCH), lambda k: (0, 0)),
        out_shape=jax.ShapeDtypeStruct((OUT_DIM, BATCH), jnp.float32),
        scratch_shapes=[
            pltpu.VMEM((OUT_DIM, BATCH), jnp.float32),
            pltpu.VMEM((OUT_DIM, IN_DIM), jnp.float32),
            pltpu.VMEM((OUT_DIM, IN_DIM), jnp.float32),
            pltpu.VMEM((IN_DIM, OUT_DIM), jnp.bfloat16),
            pltpu.SemaphoreType.DMA,
            pltpu.SemaphoreType.DMA,
        ],
    )(xt, xt, alphas, gumbel)
    return (jnp.transpose(out_t), None)


# speedup vs baseline: 1.0030x; 1.0030x over previous
import jax
import jax.numpy as jnp
from jax.experimental import pallas as pl
from jax.experimental.pallas import tpu as pltpu

OUT_DIM = 32
IN_DIM = 50000
BATCH = 4096
_TAU = 10.0 * (0.01 / 10.0) ** (1.0 / 10000.0)

KB = 1000
KSTEPS = IN_DIM // KB

CH = 5120
_CHUNKS = [(o, min(CH, IN_DIM - o)) for o in range(0, IN_DIM, CH)]


def _fused_kernel(xt_ref, a_hbm, g_hbm, out_ref,
                  acc_ref, abuf, gbuf, mt_ref, sema, semg):
    k = pl.program_id(0)

    @pl.when(k == 0)
    def _softmax():
        cpa = pltpu.make_async_copy(a_hbm, abuf, sema)
        cpg = pltpu.make_async_copy(g_hbm, gbuf, semg)
        cpa.start()
        cpg.start()
        cpa.wait()
        cpg.wait()
        s = jnp.zeros((OUT_DIM, 1), jnp.float32)
        for off, w in _CHUNKS:
            e = jnp.exp((abuf[:, off:off + w] + gbuf[:, off:off + w])
                        * (1.0 / _TAU))
            abuf[:, off:off + w] = e
            s = s + jnp.sum(e, axis=1, keepdims=True)
        rs = 1.0 / s
        for off, w in _CHUNKS:
            mt_ref[off:off + w, :] = (
                abuf[:, off:off + w] * rs).astype(jnp.bfloat16).T

    m = mt_ref[pl.ds(k * KB, KB), :].T
    part = jnp.dot(m, xt_ref[...].astype(jnp.bfloat16),
                   preferred_element_type=jnp.float32)

    @pl.when(k == 0)
    def _init():
        acc_ref[...] = part

    @pl.when(k != 0)
    def _acc():
        acc_ref[...] += part

    @pl.when(k == KSTEPS - 1)
    def _fin():
        out_ref[...] = acc_ref[...]


def kernel(x, alphas, gumbel):
    xt = jnp.transpose(x)

    out_t = pl.pallas_call(
        _fused_kernel,
        grid=(KSTEPS,),
        in_specs=[
            pl.BlockSpec((KB, BATCH), lambda k: (k, 0)),
            pl.BlockSpec(memory_space=pltpu.HBM),
            pl.BlockSpec(memory_space=pltpu.HBM),
        ],
        out_specs=pl.BlockSpec((OUT_DIM, BATCH), lambda k: (0, 0)),
        out_shape=jax.ShapeDtypeStruct((OUT_DIM, BATCH), jnp.float32),
        scratch_shapes=[
            pltpu.VMEM((OUT_DIM, BATCH), jnp.float32),
            pltpu.VMEM((OUT_DIM, IN_DIM), jnp.float32),
            pltpu.VMEM((OUT_DIM, IN_DIM), jnp.float32),
            pltpu.VMEM((IN_DIM, OUT_DIM), jnp.bfloat16),
            pltpu.SemaphoreType.DMA,
            pltpu.SemaphoreType.DMA,
        ],
    )(xt, alphas, gumbel)
    return (jnp.transpose(out_t), None)
